# baseline (speedup 1.0000x reference)
import functools

import jax
import jax.numpy as jnp
from jax.experimental import pallas as pl
from jax.experimental.pallas import tpu as pltpu

_MIB = 1024 * 1024


def _se_pipe(x_hbm, w1_ref, b1_ref, w2_ref, b2_ref, o_hbm,
             buf, in_sem, out_sem, *, bt, runs, inv_hw):
    base = 0
    for run in runs:
        def dma_in(j):
            pltpu.make_async_copy(
                x_hbm.at[pl.ds((base + j) * bt, bt)],
                buf.at[j], in_sem.at[j]).start()

        def wait_in(j):
            pltpu.make_async_copy(buf.at[j], buf.at[j], in_sem.at[j]).wait()

        def dma_out(j):
            pltpu.make_async_copy(
                buf.at[j],
                o_hbm.at[pl.ds((base + j) * bt, bt)], out_sem.at[j]).start()

        def wait_out(j):
            pltpu.make_async_copy(buf.at[j], buf.at[j], out_sem.at[j]).wait()

        for j in range(run):
            dma_in(j)
        for j in range(run):
            wait_in(j)
            xb = buf[j]
            s = jnp.sum(xb, axis=2, dtype=jnp.float32) * inv_hw
            h = jax.lax.dot_general(s, w1_ref[...], (((1,), (1,)), ((), ())),
                                    preferred_element_type=jnp.float32)
            h = jnp.maximum(h + b1_ref[...], 0.0)
            g = jax.lax.dot_general(h, w2_ref[...], (((1,), (1,)), ((), ())),
                                    preferred_element_type=jnp.float32)
            g = jax.nn.sigmoid(g + b2_ref[...])
            buf[j] = xb * g[:, :, None]
        for j in range(run):
            dma_out(j)
        for j in range(run):
            wait_out(j)
        base += run


@jax.jit
def kernel(x, w1, b1, w2, b2):
    B, C, H, W = x.shape
    Cr = w1.shape[0]
    HW = H * W
    f32 = jnp.float32

    x3 = x.reshape(B, C, HW)
    b1r = b1.reshape(1, Cr).astype(f32)
    b2r = b2.reshape(1, C).astype(f32)

    bt = 8
    n_chunks = B // bt
    slots = min(6, n_chunks)
    runs = []
    left = n_chunks
    while left > 0:
        r = min(slots, left)
        runs.append(r)
        left -= r
    buf_bytes = slots * bt * C * HW * jnp.dtype(x.dtype).itemsize

    out = pl.pallas_call(
        functools.partial(_se_pipe, bt=bt, runs=runs, inv_hw=1.0 / HW),
        out_shape=jax.ShapeDtypeStruct((B, C, HW), x.dtype),
        in_specs=[
            pl.BlockSpec(memory_space=pl.ANY),
            pl.BlockSpec(memory_space=pltpu.VMEM),
            pl.BlockSpec(memory_space=pltpu.VMEM),
            pl.BlockSpec(memory_space=pltpu.VMEM),
            pl.BlockSpec(memory_space=pltpu.VMEM),
        ],
        out_specs=pl.BlockSpec(memory_space=pl.ANY),
        scratch_shapes=[
            pltpu.VMEM((slots, bt, C, HW), x.dtype),
            pltpu.SemaphoreType.DMA((slots,)),
            pltpu.SemaphoreType.DMA((slots,)),
        ],
        compiler_params=pltpu.CompilerParams(
            vmem_limit_bytes=buf_bytes + 8 * _MIB,
        ),
    )(x3, w1.astype(f32), b1r, w2.astype(f32), b2r)
    return out.reshape(B, C, H, W)


# speedup vs baseline: 1.0153x; 1.0153x over previous
import jax
import jax.numpy as jnp
from jax.experimental import pallas as pl
from jax.experimental.pallas import tpu as pltpu

_MIB = 1024 * 1024


def _se_body(x_ref, w1_ref, b1_ref, w2_ref, b2_ref, o_ref, *, inv_hw):
    s = jnp.sum(x_ref[...], axis=2, dtype=jnp.float32) * inv_hw
    h = jax.lax.dot_general(s, w1_ref[...], (((1,), (1,)), ((), ())),
                            preferred_element_type=jnp.float32)
    h = jnp.maximum(h + b1_ref[...], 0.0)
    g = jax.lax.dot_general(h, w2_ref[...], (((1,), (1,)), ((), ())),
                            preferred_element_type=jnp.float32)
    g = jax.nn.sigmoid(g + b2_ref[...])
    o_ref[...] = (x_ref[...] * g.astype(x_ref.dtype)[:, :, None]).astype(o_ref.dtype)


def _pick_bt(B, tile_bytes, budget_bytes):
    fits = [d for d in range(B, 0, -1) if B % d == 0 and 4 * d * tile_bytes <= budget_bytes]
    small = [d for d in fits if B // d >= 4]
    return (small or fits)[0] if fits else 1


@jax.jit
def kernel(x, w1, b1, w2, b2):
    B, C, H, W = x.shape
    Cr = w1.shape[0]
    HW = H * W
    f32 = jnp.float32

    x3 = x.reshape(B, C, HW)
    b1r = b1.reshape(1, Cr).astype(f32)
    b2r = b2.reshape(1, C).astype(f32)

    tile_bytes = C * HW * jnp.dtype(x.dtype).itemsize
    bt = _pick_bt(B, tile_bytes, 36 * _MIB)

    import functools
    out = pl.pallas_call(
        functools.partial(_se_body, inv_hw=1.0 / HW),
        out_shape=jax.ShapeDtypeStruct((B, C, HW), x.dtype),
        grid=(B // bt,),
        in_specs=[
            pl.BlockSpec((bt, C, HW), lambda i: (i, 0, 0)),
            pl.BlockSpec((Cr, C), lambda i: (0, 0)),
            pl.BlockSpec((1, Cr), lambda i: (0, 0)),
            pl.BlockSpec((C, Cr), lambda i: (0, 0)),
            pl.BlockSpec((1, C), lambda i: (0, 0)),
        ],
        out_specs=pl.BlockSpec((bt, C, HW), lambda i: (i, 0, 0)),
        compiler_params=pltpu.CompilerParams(
            dimension_semantics=("parallel",),
            vmem_limit_bytes=4 * bt * tile_bytes + 8 * _MIB,
        ),
    )(x3, w1.astype(f32), b1r, w2.astype(f32), b2r)
    return out.reshape(B, C, H, W)


# speedup vs baseline: 1.0213x; 1.0059x over previous
import functools

import jax
import jax.numpy as jnp
from jax.experimental import pallas as pl
from jax.experimental.pallas import tpu as pltpu

_MIB = 1024 * 1024


def _se_pipe(x_hbm, w1_ref, b1_ref, w2_ref, b2_ref, o_hbm,
             x_buf, o_buf, in_sem, out_sem, *, bt, n_steps, inv_hw):
    def dma_in(i):
        pltpu.make_async_copy(x_hbm.at[pl.ds(i * bt, bt)],
                              x_buf.at[i % 2], in_sem.at[i % 2]).start()

    def wait_in(i):
        pltpu.make_async_copy(x_buf.at[i % 2], x_buf.at[i % 2],
                              in_sem.at[i % 2]).wait()

    def dma_out(i):
        pltpu.make_async_copy(o_buf.at[i % 2],
                              o_hbm.at[pl.ds(i * bt, bt)],
                              out_sem.at[i % 2]).start()

    def wait_out(i):
        pltpu.make_async_copy(o_buf.at[i % 2], o_buf.at[i % 2],
                              out_sem.at[i % 2]).wait()

    dma_in(0)
    for i in range(n_steps):
        if i + 1 < n_steps:
            dma_in(i + 1)
        wait_in(i)
        if i >= 2:
            wait_out(i - 2)
        xb = x_buf[i % 2]
        s = jnp.sum(xb, axis=2, dtype=jnp.float32) * inv_hw
        h = jax.lax.dot_general(s, w1_ref[...], (((1,), (1,)), ((), ())),
                                preferred_element_type=jnp.float32)
        h = jnp.maximum(h + b1_ref[...], 0.0)
        g = jax.lax.dot_general(h, w2_ref[...], (((1,), (1,)), ((), ())),
                                preferred_element_type=jnp.float32)
        g = jax.nn.sigmoid(g + b2_ref[...])
        o_buf[i % 2] = xb * g.astype(xb.dtype)[:, :, None]
        dma_out(i)
    wait_out(n_steps - 2)
    wait_out(n_steps - 1)


def _pick_bt(B, tile_bytes, budget_bytes):
    fits = [d for d in range(B, 0, -1) if B % d == 0 and 4 * d * tile_bytes <= budget_bytes]
    small = [d for d in fits if B // d >= 4]
    return (small or fits)[0] if fits else 1


@jax.jit
def kernel(x, w1, b1, w2, b2):
    B, C, H, W = x.shape
    Cr = w1.shape[0]
    HW = H * W
    f32 = jnp.float32

    x3 = x.reshape(B, C, HW)
    b1r = b1.reshape(1, Cr).astype(f32)
    b2r = b2.reshape(1, C).astype(f32)

    tile_bytes = C * HW * jnp.dtype(x.dtype).itemsize
    bt = _pick_bt(B, tile_bytes, 36 * _MIB)
    n_steps = B // bt
    buf_bytes = 4 * bt * tile_bytes

    out = pl.pallas_call(
        functools.partial(_se_pipe, bt=bt, n_steps=n_steps, inv_hw=1.0 / HW),
        out_shape=jax.ShapeDtypeStruct((B, C, HW), x.dtype),
        in_specs=[
            pl.BlockSpec(memory_space=pl.ANY),
            pl.BlockSpec(memory_space=pltpu.VMEM),
            pl.BlockSpec(memory_space=pltpu.VMEM),
            pl.BlockSpec(memory_space=pltpu.VMEM),
            pl.BlockSpec(memory_space=pltpu.VMEM),
        ],
        out_specs=pl.BlockSpec(memory_space=pl.ANY),
        scratch_shapes=[
            pltpu.VMEM((2, bt, C, HW), x.dtype),
            pltpu.VMEM((2, bt, C, HW), x.dtype),
            pltpu.SemaphoreType.DMA((2,)),
            pltpu.SemaphoreType.DMA((2,)),
        ],
        compiler_params=pltpu.CompilerParams(
            vmem_limit_bytes=buf_bytes + 8 * _MIB,
        ),
    )(x3, w1.astype(f32), b1r, w2.astype(f32), b2r)
    return out.reshape(B, C, H, W)
